# baseline (speedup 1.0000x reference)
import functools

import jax
import jax.numpy as jnp
from jax import lax
from jax.experimental import pallas as pl
from jax.experimental.pallas import tpu as pltpu
from jax.experimental.pallas import tpu_sc as plsc

_N = 10000
_E = 320000
_DIN = 128
_DE = 16
_DOUT = 128

_NC = 2
_NS = 16
_NW = _NC * _NS
_NP = 10240
_NH = _NP // 2
_CH = 128
_NCHT = _E // _CH
_CPW = _NCHT // _NW
_CR = 640

_EPW = _E // _NW
_KO = 40
_NKO = _EPW // _KO

_RPT = _NP // _NW

_mesh = plsc.VectorSubcoreMesh(core_axis_name="c", subcore_axis_name="s")


def _segsum_sc(ef128, v):
    @functools.partial(
        pl.kernel,
        out_type=[
            pltpu.HBM((_NW * _NP // 8, 128), jnp.float32),
            pltpu.HBM((_NW * (_CR // 8), 128), jnp.float32),
        ],
        mesh=_mesh,
        scratch_types=[
            pltpu.VMEM((_CH,), jnp.int32),
            pltpu.VMEM((_CH // 8, 128), jnp.float32),
            pltpu.VMEM((_NH // 8, 128), jnp.float32),
            pltpu.VMEM((_CR // 8, 128), jnp.float32),
        ],
    )
    def k(ef_hbm, v_hbm, s_out, c_out, vidx, er128, acc, cacc):
        cid = lax.axis_index("c")
        sid = lax.axis_index("s")
        wid = cid * _NS + sid
        nchunks = _CPW + jnp.where(wid < _NCHT - _CPW * _NW, 1, 0)
        zrow = jnp.zeros((_DE,), jnp.float32)
        lane = lax.iota(jnp.int32, 16)

        @pl.loop(0, _CR // 8)
        def _(i):
            for t in range(8):
                cacc[i, pl.ds(16 * t, 16)] = zrow

        for p in range(2):
            off = p * _NH

            @pl.loop(0, _NH // 8)
            def _(i):
                for t in range(8):
                    acc[i, pl.ds(16 * t, 16)] = zrow

            @pl.loop(0, nchunks)
            def _(kk):
                c = wid + kk * _NW
                pltpu.sync_copy(
                    v_hbm.at[pl.ds(pl.multiple_of(c * _CH, _CH), _CH)], vidx)
                pltpu.sync_copy(
                    ef_hbm.at[pl.ds(pl.multiple_of(c * (_CH // 8), _CH // 8),
                                    _CH // 8)], er128)

                @pl.loop(0, _CH // 16)
                def _(g):
                    vv = vidx[pl.ds(g * 16, 16)]
                    for j in range(16):
                        nv = vv[j]
                        row8 = g * 2 + (j // 8)
                        lo = 16 * (j % 8)
                        erow = er128[row8, pl.ds(lo, 16)]
                        rel = nv - off
                        ok = (rel >= 0) & (rel < _NH)
                        rowc = jnp.clip(rel, 0, _NH - 1)
                        sel = jnp.where(ok, 1.0, 0.0)
                        arow = lax.shift_right_logical(rowc, 3)
                        alo = 16 * (rowc & 7)
                        asl = pl.ds(pl.multiple_of(alo, 16), 16)
                        acc[arow, asl] = acc[arow, asl] + erow * jnp.full(
                            (16,), sel, jnp.float32)
                        if p == 0:
                            cr = lax.shift_right_logical(nv, 7)
                            clo = 16 * (lax.shift_right_logical(nv, 4) & 7)
                            csl = pl.ds(pl.multiple_of(clo, 16), 16)
                            onehot = jnp.where(
                                lane == jnp.full((16,), nv & 15, jnp.int32),
                                1.0, 0.0)
                            cacc[cr, csl] = cacc[cr, csl] + onehot

            pltpu.sync_copy(
                acc,
                s_out.at[pl.ds(pl.multiple_of((wid * _NP + off) // 8, 8),
                               _NH // 8)])

        pltpu.sync_copy(
            cacc,
            c_out.at[pl.ds(pl.multiple_of(wid * (_CR // 8), 8), _CR // 8)])

    return k(ef128, v)


def _repack_tc(ef3):
    def body(x_ref, o_ref):
        x = x_ref[...][:, 0, :]
        x3 = jnp.reshape(x, (1600, 8, _DE))
        o_ref[...] = jnp.concatenate(
            [x3[:, t, :] for t in range(8)], axis=1)

    return pl.pallas_call(
        body,
        grid=(25,),
        in_specs=[pl.BlockSpec((12800, 1, _DE), lambda i: (i, 0, 0))],
        out_specs=pl.BlockSpec((1600, 128), lambda i: (i, 0)),
        out_shape=jax.ShapeDtypeStruct((_E // 8, 128), jnp.float32),
    )(ef3)


def _reduce_tc(s_part32):
    def body(sp_ref, o_ref):
        w = jnp.sum(sp_ref[...], axis=0)
        x = jnp.stack([w[:, 16 * t:16 * (t + 1)] for t in range(8)], axis=1)
        o_ref[...] = jnp.reshape(x, (1024, _DE))

    return pl.pallas_call(
        body,
        grid=(_NP // 1024,),
        in_specs=[pl.BlockSpec((_NW, 128, 128), lambda i: (0, i, 0))],
        out_specs=pl.BlockSpec((1024, _DE), lambda i: (i, 0)),
        out_shape=jax.ShapeDtypeStruct((_NP, _DE), jnp.float32),
    )(s_part32)


def _reduce_sc(s_part):
    @functools.partial(
        pl.kernel,
        out_type=pltpu.HBM((_NP, _DE), jnp.float32),
        mesh=_mesh,
        scratch_types=[
            pltpu.VMEM((_RPT // 8, 128), jnp.float32),
            pltpu.VMEM((_RPT // 8, 128), jnp.float32),
            pltpu.VMEM((_RPT, _DE), jnp.float32),
        ],
    )
    def k(s_hbm, s_sum, accw, buf, nrw):
        cid = lax.axis_index("c")
        sid = lax.axis_index("s")
        wid = cid * _NS + sid
        nrows = _RPT // 8

        zrow = jnp.zeros((16,), jnp.float32)

        @pl.loop(0, nrows)
        def _(i):
            for t in range(8):
                accw[i, pl.ds(16 * t, 16)] = zrow

        @pl.loop(0, _NW)
        def _(q):
            r0 = pl.multiple_of(q * (_NP // 8) + wid * nrows, 8)
            pltpu.sync_copy(s_hbm.at[pl.ds(r0, nrows)], buf)

            @pl.loop(0, nrows)
            def _(i):
                for t in range(8):
                    sl = pl.ds(16 * t, 16)
                    accw[i, sl] = accw[i, sl] + buf[i, sl]

        @pl.loop(0, nrows)
        def _(i):
            for t in range(8):
                nrw[i * 8 + t, :] = accw[i, pl.ds(16 * t, 16)]

        pltpu.sync_copy(
            nrw, s_sum.at[pl.ds(pl.multiple_of(wid * _RPT, 8), _RPT)])

    return k(s_part)


_BLK = 1000


def _apply_tc(nf2, s, cnt2, wnt, wet, w1t, w2t, ba, be):
    def body(nf_ref, s_ref, c_ref, wn_ref, we_ref, w1_ref, w2_ref,
             ba_ref, be_ref, h_ref, a_ref, b_ref):
        hi = jax.lax.Precision.HIGHEST
        hn = s_ref[...] / jnp.maximum(c_ref[...], 1.0)
        h = (jnp.dot(nf_ref[...], wn_ref[...], precision=hi)
             + jnp.dot(hn, we_ref[...], precision=hi) + ba_ref[...])
        h = jnp.maximum(h, 0.0)
        h_ref[...] = h
        a_ref[...] = jnp.dot(h, w1_ref[...], precision=hi)
        b_ref[...] = jnp.dot(h, w2_ref[...], precision=hi) + be_ref[...]

    return pl.pallas_call(
        body,
        grid=(_N // _BLK,),
        in_specs=[
            pl.BlockSpec((_BLK, _DIN), lambda i: (i, 0)),
            pl.BlockSpec((_BLK, _DE), lambda i: (i, 0)),
            pl.BlockSpec((_BLK, _DE), lambda i: (i, 0)),
            pl.BlockSpec((_DIN, _DOUT), lambda i: (0, 0)),
            pl.BlockSpec((_DE, _DOUT), lambda i: (0, 0)),
            pl.BlockSpec((_DOUT, 2 * _DOUT), lambda i: (0, 0)),
            pl.BlockSpec((_DOUT, 2 * _DOUT), lambda i: (0, 0)),
            pl.BlockSpec((1, _DOUT), lambda i: (0, 0)),
            pl.BlockSpec((1, 2 * _DOUT), lambda i: (0, 0)),
        ],
        out_specs=[
            pl.BlockSpec((_BLK, _DOUT), lambda i: (i, 0)),
            pl.BlockSpec((_BLK, 2 * _DOUT), lambda i: (i, 0)),
            pl.BlockSpec((_BLK, 2 * _DOUT), lambda i: (i, 0)),
        ],
        out_shape=[
            jax.ShapeDtypeStruct((_N, _DOUT), jnp.float32),
            jax.ShapeDtypeStruct((_N, 2 * _DOUT), jnp.float32),
            jax.ShapeDtypeStruct((_N, 2 * _DOUT), jnp.float32),
        ],
    )(nf2, s, cnt2, wnt, wet, w1t, w2t, ba, be)


def _edge_sc(a, bb, u, v):
    d = 2 * _DOUT

    @functools.partial(
        pl.kernel,
        out_type=jax.ShapeDtypeStruct((_E, d), jnp.float32),
        mesh=_mesh,
        scratch_types=[
            pltpu.VMEM((_KO,), jnp.int32),
            pltpu.VMEM((_KO,), jnp.int32),
            pltpu.VMEM((_KO,), jnp.int32),
            pltpu.VMEM((_KO,), jnp.int32),
            pltpu.VMEM((_KO, d), jnp.float32),
            pltpu.VMEM((_KO, d), jnp.float32),
            pltpu.VMEM((_KO, d), jnp.float32),
            pltpu.VMEM((_KO, d), jnp.float32),
            pltpu.SemaphoreType.DMA,
            pltpu.SemaphoreType.DMA,
            pltpu.SemaphoreType.DMA,
            pltpu.SemaphoreType.DMA,
            pltpu.SemaphoreType.DMA,
            pltpu.SemaphoreType.DMA,
        ],
    )
    def k(a_hbm, b_hbm, u_hbm, v_hbm, out_hbm,
          u0, u1, v0, v1, a0, a1, b0, b1,
          si0, si1, sg0, sg1, sw0, sw1):
        uidx = (u0, u1)
        vidx = (v0, v1)
        bufa = (a0, a1)
        bufb = (b0, b1)
        si = (si0, si1)
        sg = (sg0, sg1)
        sw = (sw0, sw1)
        cid = lax.axis_index("c")
        sid = lax.axis_index("s")
        wid = cid * _NS + sid
        e0 = wid * _EPW

        def idx_load(j, b):
            base = pl.multiple_of(e0 + j * _KO, _KO)
            pltpu.async_copy(u_hbm.at[pl.ds(base, _KO)], uidx[b], si[b])
            pltpu.async_copy(v_hbm.at[pl.ds(base, _KO)], vidx[b], si[b])

        def idx_drain(b):
            pltpu.make_async_copy(u_hbm.at[pl.ds(0, _KO)], uidx[b],
                                  si[b]).wait()
            pltpu.make_async_copy(v_hbm.at[pl.ds(0, _KO)], vidx[b],
                                  si[b]).wait()

        def gather_issue(b):
            pltpu.async_copy(a_hbm.at[uidx[b]], bufa[b], sg[b])
            pltpu.async_copy(b_hbm.at[vidx[b]], bufb[b], sg[b])

        def gather_drain(b):
            pltpu.make_async_copy(a_hbm.at[uidx[b]], bufa[b], sg[b]).wait()
            pltpu.make_async_copy(b_hbm.at[vidx[b]], bufb[b], sg[b]).wait()

        def write_drain(b):
            pltpu.make_async_copy(bufa[b], out_hbm.at[pl.ds(0, _KO)],
                                  sw[b]).wait()

        idx_load(0, 0)
        idx_load(1, 1)
        idx_drain(0)
        gather_issue(0)

        @pl.loop(0, _NKO // 2)
        def _(jj):
            for b in range(2):
                j = jj * 2 + b
                nb = 1 - b
                gather_drain(b)

                @pl.when(j + 1 < _NKO)
                def _():
                    idx_drain(nb)

                    @pl.when(j >= 1)
                    def _():
                        write_drain(nb)

                    gather_issue(nb)

                @pl.loop(0, _KO)
                def _(r):
                    for c0 in range(0, d, 16):
                        sl = pl.ds(c0, 16)
                        bufa[b][r, sl] = bufa[b][r, sl] + bufb[b][r, sl]

                base = pl.multiple_of(e0 + j * _KO, _KO)
                pltpu.async_copy(bufa[b], out_hbm.at[pl.ds(base, _KO)],
                                 sw[b])

                @pl.when(j + 2 < _NKO)
                def _():
                    idx_load(j + 2, b)

        write_drain(0)
        write_drain(1)

    return k(a, bb, u, v)


def kernel(nfeats, efeats, edge_index, W_apply, b_apply, W_edge, b_edge):
    nf2 = nfeats[:, 0, :]
    ef128 = _repack_tc(efeats)
    u = edge_index[0]
    v = edge_index[1]

    s_part, c_part = _segsum_sc(ef128, v)
    s_sum = _reduce_tc(s_part.reshape(_NW, _NP // 8, 128))

    cnt = jnp.sum(c_part.reshape(_NW, _CR * 16), axis=0)[:_N]
    cnt2 = jnp.broadcast_to(cnt[:, None], (_N, _DE))
    s = s_sum[:_N]

    wnt = W_apply[:, :_DIN].T
    wet = W_apply[:, _DIN:].T
    w1t = W_edge[:, :_DOUT].T
    w2t = W_edge[:, _DOUT:].T
    h2, a, bb = _apply_tc(nf2, s, cnt2, wnt, wet, w1t, w2t,
                          b_apply.reshape(1, -1), b_edge.reshape(1, -1))

    edge2 = _edge_sc(a, bb, u, v)
    return h2[:, None, :], edge2[:, None, :]


# speedup vs baseline: 1.5002x; 1.5002x over previous
import functools

import jax
import jax.numpy as jnp
from jax import lax
from jax.experimental import pallas as pl
from jax.experimental.pallas import tpu as pltpu
from jax.experimental.pallas import tpu_sc as plsc

_N = 10000
_E = 320000
_DIN = 128
_DE = 16
_DOUT = 128

_NC = 2
_NS = 16
_NW = _NC * _NS
_NP = 10240
_NH = _NP // 2
_CH = 128
_NCHT = _E // _CH
_CPW = _NCHT // _NW
_CR = 640

_EPW = _E // _NW
_KO = 40
_NKO = _EPW // _KO

_RPT = _NP // _NW

_mesh = plsc.VectorSubcoreMesh(core_axis_name="c", subcore_axis_name="s")


def _segsum_sc(ef128, v):
    @functools.partial(
        pl.kernel,
        out_type=[
            pltpu.HBM((_NW * _NP // 8, 128), jnp.float32),
            pltpu.HBM((_NW * (_CR // 8), 128), jnp.float32),
        ],
        mesh=_mesh,
        scratch_types=[
            pltpu.VMEM((_CH,), jnp.int32),
            pltpu.VMEM((_CH,), jnp.int32),
            pltpu.VMEM((_CH // 8, 128), jnp.float32),
            pltpu.VMEM((_CH // 8, 128), jnp.float32),
            pltpu.VMEM((_NH // 8, 128), jnp.float32),
            pltpu.VMEM((_CR // 8, 128), jnp.float32),
            pltpu.SemaphoreType.DMA,
            pltpu.SemaphoreType.DMA,
        ],
    )
    def k(ef_hbm, v_hbm, s_out, c_out, vx0, vx1, er0, er1, acc, cacc,
          sl0, sl1):
        vidx = (vx0, vx1)
        er128 = (er0, er1)
        sl = (sl0, sl1)
        cid = lax.axis_index("c")
        sid = lax.axis_index("s")
        wid = cid * _NS + sid
        nchunks = _CPW + jnp.where(wid < _NCHT - _CPW * _NW, 1, 0)
        zrow = jnp.zeros((_DE,), jnp.float32)
        lane = lax.iota(jnp.int32, 16)
        niter = _CPW + 2

        def chunk_of(kk):
            return jnp.minimum(wid + kk * _NW, _NCHT - 1)

        def issue_loads(kk, b):
            c = chunk_of(kk)
            pltpu.async_copy(
                v_hbm.at[pl.ds(pl.multiple_of(c * _CH, _CH), _CH)],
                vidx[b], sl[b])
            pltpu.async_copy(
                ef_hbm.at[pl.ds(pl.multiple_of(c * (_CH // 8), _CH // 8),
                                _CH // 8)], er128[b], sl[b])

        def drain_loads(b):
            pltpu.make_async_copy(v_hbm.at[pl.ds(0, _CH)], vidx[b],
                                  sl[b]).wait()
            pltpu.make_async_copy(ef_hbm.at[pl.ds(0, _CH // 8)], er128[b],
                                  sl[b]).wait()

        @pl.loop(0, _CR // 8)
        def _(i):
            for t in range(8):
                cacc[i, pl.ds(16 * t, 16)] = zrow

        for p in range(2):
            off = p * _NH

            @pl.loop(0, _NH // 8)
            def _(i):
                for t in range(8):
                    acc[i, pl.ds(16 * t, 16)] = zrow

            issue_loads(0, 0)

            @pl.loop(0, niter // 2)
            def _(jj):
                for b in range(2):
                    kk = jj * 2 + b
                    drain_loads(b)

                    @pl.when(kk + 1 < niter)
                    def _():
                        issue_loads(kk + 1, 1 - b)

                    scsel = jnp.where(kk < nchunks, 1.0, 0.0)

                    @pl.loop(0, _CH // 16)
                    def _(g):
                        vv = vidx[b][pl.ds(g * 16, 16)]
                        for j in range(16):
                            nv = vv[j]
                            row8 = g * 2 + (j // 8)
                            lo = 16 * (j % 8)
                            erow = er128[b][row8, pl.ds(lo, 16)]
                            rel = nv - off
                            ok = (rel >= 0) & (rel < _NH)
                            rowc = jnp.clip(rel, 0, _NH - 1)
                            sel = jnp.where(ok, scsel, 0.0)
                            arow = lax.shift_right_logical(rowc, 3)
                            alo = 16 * (rowc & 7)
                            asl = pl.ds(pl.multiple_of(alo, 16), 16)
                            plsc.addupdate(
                                acc.at[arow, asl],
                                erow * jnp.full((16,), sel, jnp.float32))
                            if p == 0:
                                cr = lax.shift_right_logical(nv, 7)
                                clo = 16 * (lax.shift_right_logical(nv, 4)
                                            & 7)
                                csl = pl.ds(pl.multiple_of(clo, 16), 16)
                                onehot = jnp.where(
                                    lane == jnp.full((16,), nv & 15,
                                                     jnp.int32),
                                    scsel, 0.0)
                                plsc.addupdate(cacc.at[cr, csl], onehot)

            pltpu.sync_copy(
                acc,
                s_out.at[pl.ds(pl.multiple_of((wid * _NP + off) // 8, 8),
                               _NH // 8)])

        pltpu.sync_copy(
            cacc,
            c_out.at[pl.ds(pl.multiple_of(wid * (_CR // 8), 8), _CR // 8)])

    return k(ef128, v)


def _repack_tc(ef3):
    def body(x_ref, o_ref):
        x = x_ref[...][:, 0, :]
        x3 = jnp.reshape(x, (1600, 8, _DE))
        o_ref[...] = jnp.concatenate(
            [x3[:, t, :] for t in range(8)], axis=1)

    return pl.pallas_call(
        body,
        grid=(25,),
        in_specs=[pl.BlockSpec((12800, 1, _DE), lambda i: (i, 0, 0))],
        out_specs=pl.BlockSpec((1600, 128), lambda i: (i, 0)),
        out_shape=jax.ShapeDtypeStruct((_E // 8, 128), jnp.float32),
    )(ef3)


def _reduce_tc(s_part32):
    def body(sp_ref, o_ref):
        w = jnp.sum(sp_ref[...], axis=0)
        x = jnp.stack([w[:, 16 * t:16 * (t + 1)] for t in range(8)], axis=1)
        o_ref[...] = jnp.reshape(x, (1024, _DE))

    return pl.pallas_call(
        body,
        grid=(_NP // 1024,),
        in_specs=[pl.BlockSpec((_NW, 128, 128), lambda i: (0, i, 0))],
        out_specs=pl.BlockSpec((1024, _DE), lambda i: (i, 0)),
        out_shape=jax.ShapeDtypeStruct((_NP, _DE), jnp.float32),
    )(s_part32)


def _reduce_sc(s_part):
    @functools.partial(
        pl.kernel,
        out_type=pltpu.HBM((_NP, _DE), jnp.float32),
        mesh=_mesh,
        scratch_types=[
            pltpu.VMEM((_RPT // 8, 128), jnp.float32),
            pltpu.VMEM((_RPT // 8, 128), jnp.float32),
            pltpu.VMEM((_RPT, _DE), jnp.float32),
        ],
    )
    def k(s_hbm, s_sum, accw, buf, nrw):
        cid = lax.axis_index("c")
        sid = lax.axis_index("s")
        wid = cid * _NS + sid
        nrows = _RPT // 8

        zrow = jnp.zeros((16,), jnp.float32)

        @pl.loop(0, nrows)
        def _(i):
            for t in range(8):
                accw[i, pl.ds(16 * t, 16)] = zrow

        @pl.loop(0, _NW)
        def _(q):
            r0 = pl.multiple_of(q * (_NP // 8) + wid * nrows, 8)
            pltpu.sync_copy(s_hbm.at[pl.ds(r0, nrows)], buf)

            @pl.loop(0, nrows)
            def _(i):
                for t in range(8):
                    sl = pl.ds(16 * t, 16)
                    accw[i, sl] = accw[i, sl] + buf[i, sl]

        @pl.loop(0, nrows)
        def _(i):
            for t in range(8):
                nrw[i * 8 + t, :] = accw[i, pl.ds(16 * t, 16)]

        pltpu.sync_copy(
            nrw, s_sum.at[pl.ds(pl.multiple_of(wid * _RPT, 8), _RPT)])

    return k(s_part)


_BLK = 1000


def _apply_tc(nf2, s, cnt2, wnt, wet, w1t, w2t, ba, be):
    def body(nf_ref, s_ref, c_ref, wn_ref, we_ref, w1_ref, w2_ref,
             ba_ref, be_ref, h_ref, a_ref, b_ref):
        hi = jax.lax.Precision.HIGHEST
        hn = s_ref[...] / jnp.maximum(c_ref[...], 1.0)
        h = (jnp.dot(nf_ref[...], wn_ref[...], precision=hi)
             + jnp.dot(hn, we_ref[...], precision=hi) + ba_ref[...])
        h = jnp.maximum(h, 0.0)
        h_ref[...] = h
        a_ref[...] = jnp.dot(h, w1_ref[...], precision=hi)
        b_ref[...] = jnp.dot(h, w2_ref[...], precision=hi) + be_ref[...]

    return pl.pallas_call(
        body,
        grid=(_N // _BLK,),
        in_specs=[
            pl.BlockSpec((_BLK, _DIN), lambda i: (i, 0)),
            pl.BlockSpec((_BLK, _DE), lambda i: (i, 0)),
            pl.BlockSpec((_BLK, _DE), lambda i: (i, 0)),
            pl.BlockSpec((_DIN, _DOUT), lambda i: (0, 0)),
            pl.BlockSpec((_DE, _DOUT), lambda i: (0, 0)),
            pl.BlockSpec((_DOUT, 2 * _DOUT), lambda i: (0, 0)),
            pl.BlockSpec((_DOUT, 2 * _DOUT), lambda i: (0, 0)),
            pl.BlockSpec((1, _DOUT), lambda i: (0, 0)),
            pl.BlockSpec((1, 2 * _DOUT), lambda i: (0, 0)),
        ],
        out_specs=[
            pl.BlockSpec((_BLK, _DOUT), lambda i: (i, 0)),
            pl.BlockSpec((_BLK, 2 * _DOUT), lambda i: (i, 0)),
            pl.BlockSpec((_BLK, 2 * _DOUT), lambda i: (i, 0)),
        ],
        out_shape=[
            jax.ShapeDtypeStruct((_N, _DOUT), jnp.float32),
            jax.ShapeDtypeStruct((_N, 2 * _DOUT), jnp.float32),
            jax.ShapeDtypeStruct((_N, 2 * _DOUT), jnp.float32),
        ],
    )(nf2, s, cnt2, wnt, wet, w1t, w2t, ba, be)


def _edge_sc(a, bb, u, v):
    d = 2 * _DOUT

    @functools.partial(
        pl.kernel,
        out_type=jax.ShapeDtypeStruct((_E, d), jnp.float32),
        mesh=_mesh,
        scratch_types=[
            pltpu.VMEM((_KO,), jnp.int32),
            pltpu.VMEM((_KO,), jnp.int32),
            pltpu.VMEM((_KO,), jnp.int32),
            pltpu.VMEM((_KO,), jnp.int32),
            pltpu.VMEM((_KO, d), jnp.float32),
            pltpu.VMEM((_KO, d), jnp.float32),
            pltpu.VMEM((_KO, d), jnp.float32),
            pltpu.VMEM((_KO, d), jnp.float32),
            pltpu.SemaphoreType.DMA,
            pltpu.SemaphoreType.DMA,
            pltpu.SemaphoreType.DMA,
            pltpu.SemaphoreType.DMA,
            pltpu.SemaphoreType.DMA,
            pltpu.SemaphoreType.DMA,
        ],
    )
    def k(a_hbm, b_hbm, u_hbm, v_hbm, out_hbm,
          u0, u1, v0, v1, a0, a1, b0, b1,
          si0, si1, sg0, sg1, sw0, sw1):
        uidx = (u0, u1)
        vidx = (v0, v1)
        bufa = (a0, a1)
        bufb = (b0, b1)
        si = (si0, si1)
        sg = (sg0, sg1)
        sw = (sw0, sw1)
        cid = lax.axis_index("c")
        sid = lax.axis_index("s")
        wid = cid * _NS + sid
        e0 = wid * _EPW

        def idx_load(j, b):
            base = pl.multiple_of(e0 + j * _KO, _KO)
            pltpu.async_copy(u_hbm.at[pl.ds(base, _KO)], uidx[b], si[b])
            pltpu.async_copy(v_hbm.at[pl.ds(base, _KO)], vidx[b], si[b])

        def idx_drain(b):
            pltpu.make_async_copy(u_hbm.at[pl.ds(0, _KO)], uidx[b],
                                  si[b]).wait()
            pltpu.make_async_copy(v_hbm.at[pl.ds(0, _KO)], vidx[b],
                                  si[b]).wait()

        def gather_issue(b):
            pltpu.async_copy(a_hbm.at[uidx[b]], bufa[b], sg[b])
            pltpu.async_copy(b_hbm.at[vidx[b]], bufb[b], sg[b])

        def gather_drain(b):
            pltpu.make_async_copy(a_hbm.at[uidx[b]], bufa[b], sg[b]).wait()
            pltpu.make_async_copy(b_hbm.at[vidx[b]], bufb[b], sg[b]).wait()

        def write_drain(b):
            pltpu.make_async_copy(bufa[b], out_hbm.at[pl.ds(0, _KO)],
                                  sw[b]).wait()

        idx_load(0, 0)
        idx_load(1, 1)
        idx_drain(0)
        gather_issue(0)

        @pl.loop(0, _NKO // 2)
        def _(jj):
            for b in range(2):
                j = jj * 2 + b
                nb = 1 - b
                gather_drain(b)

                @pl.when(j + 1 < _NKO)
                def _():
                    idx_drain(nb)

                    @pl.when(j >= 1)
                    def _():
                        write_drain(nb)

                    gather_issue(nb)

                @pl.loop(0, _KO)
                def _(r):
                    for c0 in range(0, d, 16):
                        sl = pl.ds(c0, 16)
                        plsc.addupdate(bufa[b].at[r, sl], bufb[b][r, sl])

                base = pl.multiple_of(e0 + j * _KO, _KO)
                pltpu.async_copy(bufa[b], out_hbm.at[pl.ds(base, _KO)],
                                 sw[b])

                @pl.when(j + 2 < _NKO)
                def _():
                    idx_load(j + 2, b)

        write_drain(0)
        write_drain(1)

    return k(a, bb, u, v)


def kernel(nfeats, efeats, edge_index, W_apply, b_apply, W_edge, b_edge):
    nf2 = nfeats[:, 0, :]
    ef128 = efeats.reshape(_E // 8, 128)
    u = edge_index[0]
    v = edge_index[1]

    s_part, c_part = _segsum_sc(ef128, v)
    s_sum = _reduce_tc(s_part.reshape(_NW, _NP // 8, 128))

    cnt = jnp.sum(c_part.reshape(_NW, _CR * 16), axis=0)[:_N]
    cnt2 = jnp.broadcast_to(cnt[:, None], (_N, _DE))
    s = s_sum[:_N]

    wnt = W_apply[:, :_DIN].T
    wet = W_apply[:, _DIN:].T
    w1t = W_edge[:, :_DOUT].T
    w2t = W_edge[:, _DOUT:].T
    h2, a, bb = _apply_tc(nf2, s, cnt2, wnt, wet, w1t, w2t,
                          b_apply.reshape(1, -1), b_edge.reshape(1, -1))

    edge2 = _edge_sc(a, bb, u, v)
    return h2[:, None, :], edge2[:, None, :]
